# baseline (speedup 1.0000x reference)
import functools

import jax
import jax.numpy as jnp
from jax import lax
from jax.experimental import pallas as pl
from jax.experimental.pallas import tpu as pltpu
from jax.experimental.pallas import tpu_sc as plsc

N = 320000
D = 128
S = 10000

IDROW = 128
IDROWS = N // IDROW
IDROWS_PAD = 2504
NW = 16
UNITS = IDROWS // 8
BIGW = 8
DBROWS = 256
SEG_PER_TILE = 624
SEG_TAIL = S - 16 * SEG_PER_TILE

_mesh = plsc.VectorSubcoreMesh(core_axis_name="c", subcore_axis_name="s",
                               num_cores=1)


@functools.partial(
    pl.kernel,
    out_type=jax.ShapeDtypeStruct((S, D), jnp.float32),
    mesh=_mesh,
    scratch_types=[
        pltpu.VMEM((DBROWS, D), jnp.float32),
        pltpu.VMEM((8, IDROW), jnp.int32),
        pltpu.VMEM_SHARED((S, D), jnp.float32),
    ],
)
def _seg_sum_sc(data_hbm, ids_hbm, zeros_hbm, out_hbm, dbuf, ibuf, acc_sh):
    s = lax.axis_index("s")
    w = s

    pltpu.sync_copy(zeros_hbm, dbuf)
    base = pl.multiple_of(s * SEG_PER_TILE, 8)
    pltpu.sync_copy(dbuf, acc_sh.at[pl.ds(base, DBROWS)])
    pltpu.sync_copy(dbuf, acc_sh.at[pl.ds(base + DBROWS, DBROWS)])
    pltpu.sync_copy(dbuf.at[pl.ds(0, SEG_PER_TILE - 2 * DBROWS)],
                    acc_sh.at[pl.ds(base + 2 * DBROWS,
                                    SEG_PER_TILE - 2 * DBROWS)])

    @pl.when(s == 0)
    def _zero_tail():
        pltpu.sync_copy(dbuf.at[pl.ds(0, SEG_TAIL)],
                        acc_sh.at[pl.ds(16 * SEG_PER_TILE, SEG_TAIL)])

    plsc.subcore_barrier()

    start_unit = w * 20 - jnp.maximum(w - BIGW, 0)
    n_units = 20 - (w >= BIGW).astype(jnp.int32)

    def unit_body(u, carry):
        unit = start_unit + u
        idrow0 = pl.multiple_of(unit * 8, 8)
        pltpu.sync_copy(ids_hbm.at[pl.ds(idrow0, 8)], ibuf)
        for j in range(4):
            row0 = pl.multiple_of(unit * 1024 + j * DBROWS, 8)
            pltpu.sync_copy(data_hbm.at[pl.ds(row0, DBROWS)], dbuf)
            for h in range(2):
                pltpu.sync_copy(dbuf.at[pl.ds(h * IDROW, IDROW)],
                                acc_sh.at[ibuf.at[2 * j + h]], add=True)
        return carry

    lax.fori_loop(0, n_units, unit_body, 0)

    @pl.when(w == NW - 1)
    def _tail():
        pltpu.sync_copy(ids_hbm.at[pl.ds(UNITS * 8, 8)], ibuf)
        for j in range(2):
            row0 = pl.multiple_of(UNITS * 1024 + j * DBROWS, 8)
            pltpu.sync_copy(data_hbm.at[pl.ds(row0, DBROWS)], dbuf)
            for h in range(2):
                pltpu.sync_copy(dbuf.at[pl.ds(h * IDROW, IDROW)],
                                acc_sh.at[ibuf.at[2 * j + h]], add=True)

    plsc.subcore_barrier()

    pltpu.sync_copy(acc_sh.at[pl.ds(base, DBROWS)], dbuf)
    pltpu.sync_copy(dbuf, out_hbm.at[pl.ds(base, DBROWS)])
    pltpu.sync_copy(acc_sh.at[pl.ds(base + DBROWS, DBROWS)], dbuf)
    pltpu.sync_copy(dbuf, out_hbm.at[pl.ds(base + DBROWS, DBROWS)])
    pltpu.sync_copy(acc_sh.at[pl.ds(base + 2 * DBROWS,
                                    SEG_PER_TILE - 2 * DBROWS)],
                    dbuf.at[pl.ds(0, SEG_PER_TILE - 2 * DBROWS)])
    pltpu.sync_copy(dbuf.at[pl.ds(0, SEG_PER_TILE - 2 * DBROWS)],
                    out_hbm.at[pl.ds(base + 2 * DBROWS,
                                     SEG_PER_TILE - 2 * DBROWS)])

    @pl.when(s == 0)
    def _write_tail():
        pltpu.sync_copy(acc_sh.at[pl.ds(16 * SEG_PER_TILE, SEG_TAIL)],
                        dbuf.at[pl.ds(0, SEG_TAIL)])
        pltpu.sync_copy(dbuf.at[pl.ds(0, SEG_TAIL)],
                        out_hbm.at[pl.ds(16 * SEG_PER_TILE, SEG_TAIL)])


def kernel(data, segment_ids):
    ids2d = segment_ids.astype(jnp.int32).reshape(IDROWS, IDROW)
    ids2d = jnp.pad(ids2d, ((0, IDROWS_PAD - IDROWS), (0, 0)))
    zeros = jnp.zeros((DBROWS, D), jnp.float32)
    return _seg_sum_sc(data, ids2d, zeros)
